# Initial kernel scaffold; baseline (speedup 1.0000x reference)
import jax
import jax.numpy as jnp
from jax.experimental import pallas as pl


def kernel(offset, optical_flow):
    raise NotImplementedError("write your pallas kernel here")


# speedup vs baseline: 1.1252x; 1.1252x over previous
import jax
import jax.numpy as jnp
from jax.experimental import pallas as pl
from jax.experimental.pallas import tpu as pltpu

_OFF_HALF = 9
_N_SEG = 4
_TH = 128


def _loss_kernel(off_ref, flow_ref, out_ref):
    j_idx = pl.program_id(1)

    us, vs, invs, los, his = [], [], [], [], []
    for j in range(_N_SEG):
        u = flow_ref[0, j]
        v = flow_ref[0, j + 1]
        uu = u * u + v * v
        inv = 1.0 / uu
        lo = jnp.minimum(0.0, u) * uu
        hi = jnp.maximum(0.0, u) * uu
        lo = jnp.where(uu > 0.0, lo, 1.0)
        hi = jnp.where(uu > 0.0, hi, 0.0)
        us.append(u)
        vs.append(v)
        invs.append(inv)
        los.append(lo)
        his.append(hi)

    msum = None
    for i in range(_OFF_HALF):
        x = off_ref[0, i]
        y = off_ref[0, _OFF_HALF + i]
        xx = x * x
        d1sq = xx + y * y
        msq = None
        for j in range(_N_SEG):
            u, v, inv, lo, hi = us[j], vs[j], invs[j], los[j], his[j]
            s = u * (xx + v * y)
            inside = (lo <= s) & (s <= hi)
            t = v * x - u * y
            perpsq = t * t * inv
            dx = x - u
            dy = y - v
            d2sq = dx * dx + dy * dy
            endsq = jnp.minimum(d1sq, d2sq)
            md = jnp.where(inside, perpsq, endsq)
            msq = md if msq is None else jnp.minimum(msq, md)
        m = jnp.sqrt(msq)
        msum = m if msum is None else msum + m

    r = msum[0:8]
    for k in range(8, msum.shape[0], 8):
        r = r + msum[k:k + 8]
    w = msum.shape[1]
    r128 = r[:, 0:128]
    for k in range(128, w, 128):
        r128 = r128 + r[:, k:k + 128]
    s81 = jnp.sum(r128, axis=-1, keepdims=True)
    s11 = jnp.sum(s81, axis=0, keepdims=True)
    part = jnp.broadcast_to(s11, (8, 128))

    @pl.when(j_idx == 0)
    def _():
        out_ref[...] = jnp.zeros_like(out_ref)

    out_ref[...] += part


def kernel(offset, optical_flow):
    b, c_off, h, w = offset.shape
    of_num = optical_flow.shape[1] // 2
    flow = optical_flow[:, :of_num + 1]
    ht = h // _TH

    out = pl.pallas_call(
        _loss_kernel,
        out_shape=jax.ShapeDtypeStruct((b * 8, 128), jnp.float32),
        grid=(b, ht),
        in_specs=[
            pl.BlockSpec((1, c_off, _TH, w), lambda i, j: (i, 0, j, 0)),
            pl.BlockSpec((1, of_num + 1, _TH, w), lambda i, j: (i, 0, j, 0)),
        ],
        out_specs=pl.BlockSpec((8, 128), lambda i, j: (i, 0)),
        compiler_params=pltpu.CompilerParams(
            dimension_semantics=("parallel", "arbitrary"),
        ),
        name="position_loss_val",
    )(offset, flow)

    total = jnp.sum(out[::8, 0])
    return total / (_OFF_HALF * h * w)


# speedup vs baseline: 2.2830x; 2.0289x over previous
import jax
import jax.numpy as jnp
from jax.experimental import pallas as pl
from jax.experimental.pallas import tpu as pltpu

_OFF_HALF = 9
_N_SEG = 4
_TH = 128
_RC = 8
_CC = 128


def _loss_kernel(off_ref, flow_ref, out_ref):
    jt = pl.program_id(1)
    w = off_ref.shape[3]

    acc = None
    for r in range(0, _TH, _RC):
        for c in range(0, w, _CC):
            rs = slice(r, r + _RC)
            cs = slice(c, c + _CC)
            seg = []
            for j in range(_N_SEG):
                u = flow_ref[0, j, rs, cs]
                v = flow_ref[0, j + 1, rs, cs]
                uu = u * u + v * v
                inv = 1.0 / uu
                lo = jnp.minimum(0.0, u) * uu
                hi = jnp.maximum(0.0, u) * uu
                lo = jnp.where(uu > 0.0, lo, 1.0)
                hi = jnp.where(uu > 0.0, hi, 0.0)
                seg.append((u, v, inv, lo, hi))
            msum = None
            for i in range(_OFF_HALF):
                x = off_ref[0, i, rs, cs]
                y = off_ref[0, _OFF_HALF + i, rs, cs]
                xx = x * x
                d1sq = xx + y * y
                msq = None
                for (u, v, inv, lo, hi) in seg:
                    s = u * (xx + v * y)
                    inside = (lo <= s) & (s <= hi)
                    t = v * x - u * y
                    perpsq = t * t * inv
                    dx = x - u
                    dy = y - v
                    d2sq = dx * dx + dy * dy
                    md = jnp.where(inside, perpsq, jnp.minimum(d1sq, d2sq))
                    msq = md if msq is None else jnp.minimum(msq, md)
                m = jnp.sqrt(msq)
                msum = m if msum is None else msum + m
            acc = msum if acc is None else acc + msum

    s81 = jnp.sum(acc, axis=-1, keepdims=True)
    s11 = jnp.sum(s81, axis=0, keepdims=True)
    part = jnp.broadcast_to(s11, (_RC, _CC))

    @pl.when(jt == 0)
    def _():
        out_ref[...] = jnp.zeros_like(out_ref)

    out_ref[...] += part


def kernel(offset, optical_flow):
    b, c_off, h, w = offset.shape
    of_num = optical_flow.shape[1] // 2
    flow = optical_flow[:, :of_num + 1]
    ht = h // _TH

    out = pl.pallas_call(
        _loss_kernel,
        out_shape=jax.ShapeDtypeStruct((b * 8, 128), jnp.float32),
        grid=(b, ht),
        in_specs=[
            pl.BlockSpec((1, c_off, _TH, w), lambda i, j: (i, 0, j, 0)),
            pl.BlockSpec((1, of_num + 1, _TH, w), lambda i, j: (i, 0, j, 0)),
        ],
        out_specs=pl.BlockSpec((8, 128), lambda i, j: (i, 0)),
        compiler_params=pltpu.CompilerParams(
            dimension_semantics=("parallel", "arbitrary"),
        ),
        name="position_loss_val",
    )(offset, flow)

    total = jnp.sum(out[::8, 0])
    return total / (_OFF_HALF * h * w)
